# baseline (speedup 1.0000x reference)
import functools

import jax
import jax.numpy as jnp
from jax import lax
from jax.experimental import pallas as pl
from jax.experimental.pallas import tpu as pltpu
from jax.experimental.pallas import tpu_sc as plsc

C_IN = 100000
D_MODEL = 64

NC = 2
NS = 16
NW = NC * NS

GB = 4


def _make_gather(batch: int, seq: int):
    n_total = batch * seq
    group = GB * seq
    bat_w = batch // NW
    per_w = bat_w * seq
    n_groups = bat_w // GB
    assert batch % NW == 0 and bat_w % GB == 0 and n_groups % 2 == 0
    mesh = plsc.VectorSubcoreMesh(core_axis_name="c", subcore_axis_name="s")

    @functools.partial(
        pl.kernel,
        out_type=jax.ShapeDtypeStruct((batch, seq, D_MODEL), jnp.float32),
        mesh=mesh,
        scratch_types=[
            pltpu.VMEM((per_w,), jnp.int32),
            pltpu.VMEM((GB * seq, D_MODEL), jnp.float32),
            pltpu.VMEM((GB * seq, D_MODEL), jnp.float32),
            pltpu.SemaphoreType.DMA,
            pltpu.SemaphoreType.DMA,
            pltpu.SemaphoreType.DMA,
            pltpu.SemaphoreType.DMA,
        ],
        compiler_params=pltpu.CompilerParams(use_tc_tiling_on_sc=False),
    )
    def gather_kernel(w_hbm, x_hbm, out_hbm, idx_v, rows_a, rows_b,
                      gsem_a, gsem_b, wsem_a, wsem_b):
        wid = lax.axis_index("s") * NC + lax.axis_index("c")
        base = wid * per_w
        bat_base = wid * bat_w
        pltpu.sync_copy(x_hbm.at[pl.ds(base, per_w)], idx_v)

        def start_gather(g, rows, gsem):
            pltpu.async_copy(
                w_hbm.at[idx_v.at[pl.ds(g * group, group)]],
                rows, gsem)

        def drain_gather(rows, gsem):
            pltpu.make_async_copy(
                w_hbm.at[idx_v.at[pl.ds(0, group)]],
                rows, gsem).wait()

        def start_write(g, rows, wsem):
            for k in range(GB):
                pltpu.async_copy(rows.at[pl.ds(k * seq, seq)],
                                 out_hbm.at[bat_base + g * GB + k], wsem)

        def drain_write(rows, wsem):
            for k in range(GB):
                pltpu.make_async_copy(rows.at[pl.ds(k * seq, seq)],
                                      out_hbm.at[0], wsem).wait()

        def do_group(g, rows, gsem, wsem, nxt_rows, nxt_gsem, nxt_wsem):
            drain_gather(rows, gsem)

            @pl.when(g >= 1)
            def _():
                drain_write(nxt_rows, nxt_wsem)

            @pl.when(g + 1 < n_groups)
            def _():
                start_gather(g + 1, nxt_rows, nxt_gsem)

            start_write(g, rows, wsem)

        start_gather(0, rows_a, gsem_a)

        def pair_body(t, carry):
            do_group(2 * t, rows_a, gsem_a, wsem_a, rows_b, gsem_b, wsem_b)
            do_group(2 * t + 1, rows_b, gsem_b, wsem_b, rows_a, gsem_a, wsem_a)
            return carry

        lax.fori_loop(0, n_groups // 2, pair_body, 0)
        drain_write(rows_b, wsem_b)

    return gather_kernel


def kernel(x, W):
    b, s = x.shape
    xf = x.reshape(b * s).astype(jnp.int32)
    return _make_gather(b, s)(W, xf)
